# baseline (speedup 1.0000x reference)
import functools

import jax
import jax.numpy as jnp
from jax import lax
from jax.experimental import pallas as pl
from jax.experimental.pallas import tpu as pltpu
from jax.experimental.pallas import tpu_sc as plsc

_GAMMA = 3.0
_C = 128
_L = 16
_NC = 2
_NS = 16


def _make_dist_kernel(d, p_pad, cpw):
  mesh = plsc.VectorSubcoreMesh(
      core_axis_name="c", subcore_axis_name="s", num_cores=_NC,
      num_subcores=_NS)

  @functools.partial(
      pl.kernel,
      mesh=mesh,
      compiler_params=pltpu.CompilerParams(needs_layout_passes=False, use_tc_tiling_on_sc=False),
      out_type=jax.ShapeDtypeStruct((p_pad,), jnp.float32),
      scratch_types=[
          pltpu.VMEM((cpw, _C), jnp.int32),
          pltpu.VMEM((cpw, _C), jnp.int32),
          pltpu.VMEM((_C, d // 2), jnp.int32),
          pltpu.VMEM((_C, d // 2), jnp.int32),
          pltpu.VMEM((_C, d // 2), jnp.int32),
          pltpu.VMEM((_C, d // 2), jnp.int32),
          pltpu.VMEM((_C, _L), jnp.float32),
          pltpu.VMEM((cpw * _C,), jnp.float32),
          pltpu.SemaphoreType.DMA,
          pltpu.SemaphoreType.DMA,
          pltpu.SemaphoreType.DMA,
          pltpu.SemaphoreType.DMA,
      ],
  )
  def dist_kernel(emb, idxl, idxr, out, idxl_v, idxr_v, rl0, rr0, rl1, rr1,
                  part_v, dist_v, sl0, sr0, sl1, sr1):
    wid = lax.axis_index("s") * _NC + lax.axis_index("c")
    pltpu.sync_copy(idxl.at[wid], idxl_v)
    pltpu.sync_copy(idxr.at[wid], idxr_v)
    lane = lax.iota(jnp.int32, _L)

    def fire(t, rl, rr, sl, sr):
      pltpu.async_copy(emb.at[idxl_v.at[t]], rl, sl)
      pltpu.async_copy(emb.at[idxr_v.at[t]], rr, sr)

    def drain(t, rl, rr, sl, sr):
      pltpu.make_async_copy(emb.at[idxl_v.at[t]], rl, sl).wait()
      pltpu.make_async_copy(emb.at[idxr_v.at[t]], rr, sr).wait()

    def compute(t, rl, rr):
      def pair_body(p, carry):
        acc = jnp.zeros((_L,), jnp.float32)
        for dd in range(d // 32):
          lv = plsc.bitcast(rl[p, pl.ds(dd * _L, _L)], jnp.bfloat16)
          rv = plsc.bitcast(rr[p, pl.ds(dd * _L, _L)], jnp.bfloat16)
          df = jnp.abs(lv - rv)
          da, db = plsc.unpack(df, format=plsc.PackFormat.INTERLEAVED)
          acc = acc + da + db
        part_v[p, :] = acc
        return carry

      lax.fori_loop(0, _C, pair_body, 0)

      for g in range(_C // _L):
        row_idx = lane + (g * _L)
        acc2 = jnp.zeros((_L,), jnp.float32)
        for jj in range(_L):
          col_idx = (lane + jj) & (_L - 1)
          acc2 = acc2 + plsc.load_gather(part_v, [row_idx, col_idx])
        dist_v[pl.ds(t * _C + g * _L, _L)] = acc2

    fire(0, rl0, rr0, sl0, sr0)

    def body(tt, carry):
      t0 = 2 * tt
      t1 = t0 + 1
      fire(t1, rl1, rr1, sl1, sr1)
      drain(t0, rl0, rr0, sl0, sr0)
      compute(t0, rl0, rr0)

      @pl.when(t0 + 2 < cpw)
      def _prefetch():
        fire(t0 + 2, rl0, rr0, sl0, sr0)

      drain(t1, rl1, rr1, sl1, sr1)
      compute(t1, rl1, rr1)
      return carry

    lax.fori_loop(0, cpw // 2, body, 0)
    pltpu.sync_copy(dist_v, out.at[pl.ds(wid * (cpw * _C), cpw * _C)])

  return dist_kernel


def _hinge(a, b1, b2, t, k):
  steps = 10
  rows = t // steps
  inv = 1.0 / (2.0 * k * t)

  def body(a_ref, b1_ref, b2_ref, o_ref):
    @pl.when(pl.program_id(0) == 0)
    def _init():
      o_ref[0, 0] = 0.0

    dv = a_ref[...] + _GAMMA
    s1 = jnp.sum(jnp.maximum(dv - b1_ref[...], 0.0))
    s2 = jnp.sum(jnp.maximum(dv - b2_ref[...], 0.0))
    o_ref[0, 0] += (s1 + s2) * inv

  out = pl.pallas_call(
      body,
      grid=(steps,),
      in_specs=[
          pl.BlockSpec((rows, 1), lambda i: (i, 0)),
          pl.BlockSpec((rows, k), lambda i: (i, 0)),
          pl.BlockSpec((rows, k), lambda i: (i, 0)),
      ],
      out_specs=pl.BlockSpec((1, 1), lambda i: (0, 0),
                             memory_space=pltpu.SMEM),
      out_shape=jax.ShapeDtypeStruct((1, 1), jnp.float32),
  )(a, b1, b2)
  return out[0, 0]


def kernel(out_emb, ILL, neg_left1, neg_right1, neg_left2, neg_right2):
  n, d = out_emb.shape
  t = ILL.shape[0]
  k = neg_left1.shape[0] // t
  p = t + 2 * t * k
  nw = _NC * _NS
  cpw = -(-p // (_C * nw))
  cpw += cpw % 2
  p_pad = cpw * _C * nw
  pad = p_pad - p

  zpad = jnp.zeros((pad,), jnp.int32)
  left = jnp.concatenate([ILL[:, 0], neg_left1, neg_left2, zpad])
  right = jnp.concatenate([ILL[:, 1], neg_right1, neg_right2, zpad])
  left = left.reshape(nw, cpw, _C)
  right = right.reshape(nw, cpw, _C)

  emb16 = out_emb.astype(jnp.bfloat16).reshape(n, d // 2, 2)
  emb_i32 = lax.bitcast_convert_type(emb16, jnp.int32)
  dist = _make_dist_kernel(d, p_pad, cpw)(emb_i32, left, right)

  a = dist[:t].reshape(t, 1)
  b1 = dist[t:t + t * k].reshape(t, k)
  b2 = dist[t + t * k:t + 2 * t * k].reshape(t, k)
  return _hinge(a, b1, b2, t, k)


# speedup vs baseline: 1.2369x; 1.2369x over previous
import functools

import jax
import jax.numpy as jnp
from jax import lax
from jax.experimental import pallas as pl
from jax.experimental.pallas import tpu as pltpu
from jax.experimental.pallas import tpu_sc as plsc

_GAMMA = 3.0
_C = 128
_L = 16
_NC = 2
_NS = 16
_UNR = 8


def _make_dist_kernel(d, p_pad, cpw):
  mesh = plsc.VectorSubcoreMesh(
      core_axis_name="c", subcore_axis_name="s", num_cores=_NC,
      num_subcores=_NS)

  @functools.partial(
      pl.kernel,
      mesh=mesh,
      compiler_params=pltpu.CompilerParams(needs_layout_passes=False),
      out_type=jax.ShapeDtypeStruct((p_pad,), jnp.float32),
      scratch_types=[
          pltpu.VMEM((cpw, _C), jnp.int32),
          pltpu.VMEM((cpw, _C), jnp.int32),
          pltpu.VMEM((_C, d), jnp.float32),
          pltpu.VMEM((_C, d), jnp.float32),
          pltpu.VMEM((_C, d), jnp.float32),
          pltpu.VMEM((_C, d), jnp.float32),
          pltpu.VMEM((cpw * _C,), jnp.float32),
          pltpu.SemaphoreType.DMA,
          pltpu.SemaphoreType.DMA,
          pltpu.SemaphoreType.DMA,
          pltpu.SemaphoreType.DMA,
      ],
  )
  def dist_kernel(emb, idxl, idxr, out, idxl_v, idxr_v, rl0, rr0, rl1, rr1,
                  dist_v, sl0, sr0, sl1, sr1):
    wid = lax.axis_index("s") * _NC + lax.axis_index("c")
    pltpu.sync_copy(idxl.at[wid], idxl_v)
    pltpu.sync_copy(idxr.at[wid], idxr_v)
    lane = lax.iota(jnp.int32, _L)

    def fire(t, rl, rr, sl, sr):
      pltpu.async_copy(emb.at[idxl_v.at[t]], rl, sl)
      pltpu.async_copy(emb.at[idxr_v.at[t]], rr, sr)

    def drain(t, rl, rr, sl, sr):
      pltpu.make_async_copy(emb.at[idxl_v.at[t]], rl, sl).wait()
      pltpu.make_async_copy(emb.at[idxr_v.at[t]], rr, sr).wait()

    def compute(t, rl, rr):
      for g in range(_C // _L):
        row_idx = lane + (g * _L)

        def col_body(s, acc):
          for u in range(_UNR):
            dcol = s * _UNR + u
            col_idx = (lane + dcol) & (d - 1)
            lv = plsc.load_gather(rl, [row_idx, col_idx])
            rv = plsc.load_gather(rr, [row_idx, col_idx])
            acc = acc + jnp.abs(lv - rv)
          return acc

        acc = lax.fori_loop(0, d // _UNR, col_body,
                            jnp.zeros((_L,), jnp.float32))
        dist_v[pl.ds(t * _C + g * _L, _L)] = acc

    fire(0, rl0, rr0, sl0, sr0)

    def body(tt, carry):
      t0 = 2 * tt
      t1 = t0 + 1
      fire(t1, rl1, rr1, sl1, sr1)
      drain(t0, rl0, rr0, sl0, sr0)
      compute(t0, rl0, rr0)

      @pl.when(t0 + 2 < cpw)
      def _prefetch():
        fire(t0 + 2, rl0, rr0, sl0, sr0)

      drain(t1, rl1, rr1, sl1, sr1)
      compute(t1, rl1, rr1)
      return carry

    lax.fori_loop(0, cpw // 2, body, 0)
    pltpu.sync_copy(dist_v, out.at[pl.ds(wid * (cpw * _C), cpw * _C)])

  return dist_kernel


def _hinge(a, b1, b2, t, k):
  steps = 10
  rows = t // steps
  inv = 1.0 / (2.0 * k * t)

  def body(a_ref, b1_ref, b2_ref, o_ref):
    @pl.when(pl.program_id(0) == 0)
    def _init():
      o_ref[0, 0] = 0.0

    dv = a_ref[...] + _GAMMA
    s1 = jnp.sum(jnp.maximum(dv - b1_ref[...], 0.0))
    s2 = jnp.sum(jnp.maximum(dv - b2_ref[...], 0.0))
    o_ref[0, 0] += (s1 + s2) * inv

  out = pl.pallas_call(
      body,
      grid=(steps,),
      in_specs=[
          pl.BlockSpec((rows, 1), lambda i: (i, 0)),
          pl.BlockSpec((rows, k), lambda i: (i, 0)),
          pl.BlockSpec((rows, k), lambda i: (i, 0)),
      ],
      out_specs=pl.BlockSpec((1, 1), lambda i: (0, 0),
                             memory_space=pltpu.SMEM),
      out_shape=jax.ShapeDtypeStruct((1, 1), jnp.float32),
  )(a, b1, b2)
  return out[0, 0]


def kernel(out_emb, ILL, neg_left1, neg_right1, neg_left2, neg_right2):
  n, d = out_emb.shape
  t = ILL.shape[0]
  k = neg_left1.shape[0] // t
  p = t + 2 * t * k
  nw = _NC * _NS
  cpw = -(-p // (_C * nw))
  cpw += cpw % 2
  p_pad = cpw * _C * nw
  pad = p_pad - p

  zpad = jnp.zeros((pad,), jnp.int32)
  left = jnp.concatenate([ILL[:, 0], neg_left1, neg_left2, zpad])
  right = jnp.concatenate([ILL[:, 1], neg_right1, neg_right2, zpad])
  left = left.reshape(nw, cpw, _C)
  right = right.reshape(nw, cpw, _C)

  dist = _make_dist_kernel(d, p_pad, cpw)(out_emb, left, right)

  a = dist[:t].reshape(t, 1)
  b1 = dist[t:t + t * k].reshape(t, k)
  b2 = dist[t + t * k:t + 2 * t * k].reshape(t, k)
  return _hinge(a, b1, b2, t, k)
